# baseline (speedup 1.0000x reference)
import functools

import jax
import jax.numpy as jnp
from jax import lax
from jax.experimental import pallas as pl
from jax.experimental.pallas import tpu as pltpu
from jax.experimental.pallas import tpu_sc as plsc

H = 128
CHUNK = 128
NBUF = 3


def _sc_segment_sum(e, idx3, n_pad):
    n_chunks = idx3.shape[0] // CHUNK
    info = plsc.get_sparse_core_info()
    nc, ns = info.num_cores, info.num_subcores
    nw = nc * ns
    rows_per_tile = n_pad // ns
    n_trips = (-(-n_chunks // nw) + NBUF - 1) // NBUF

    mesh = plsc.VectorSubcoreMesh(core_axis_name="c", subcore_axis_name="s")

    @functools.partial(
        pl.kernel,
        mesh=mesh,
        out_type=jax.ShapeDtypeStruct((nc, n_pad, H), jnp.float32),
        scratch_types=[
            pltpu.VMEM((CHUNK,), jnp.int32),
            pltpu.VMEM((CHUNK,), jnp.int32),
            pltpu.VMEM((CHUNK,), jnp.int32),
            pltpu.VMEM((CHUNK, H), jnp.float32),
            pltpu.VMEM((CHUNK, H), jnp.float32),
            pltpu.VMEM((CHUNK, H), jnp.float32),
            pltpu.SemaphoreType.DMA,
            pltpu.SemaphoreType.DMA,
            pltpu.SemaphoreType.DMA,
            pltpu.VMEM_SHARED((n_pad, H), jnp.float32),
        ],
    )
    def k(e_hbm, idx_hbm, out_hbm, idx_a, idx_b, idx_c, rows_a, rows_b,
          rows_c, sem_a, sem_b, sem_c, acc_sh):
        c = lax.axis_index("c")
        s = lax.axis_index("s")
        wid = s * nc + c
        cbase = n_chunks // nw
        cextra = n_chunks - cbase * nw
        start = wid * cbase + jnp.minimum(wid, cextra)
        cnt = cbase + jnp.where(wid < cextra, 1, 0)
        idx_bufs = [idx_a, idx_b, idx_c]
        rows_bufs = [rows_a, rows_b, rows_c]
        sems = [sem_a, sem_b, sem_c]

        def fetch(ch, idx_v, rows_v, sem):
            pltpu.make_async_copy(
                idx_hbm.at[pl.ds(ch * CHUNK, CHUNK)], idx_v, sem
            ).start()
            pltpu.make_async_copy(
                e_hbm.at[pl.ds(ch * CHUNK, CHUNK)], rows_v, sem
            ).start()

        def drain_and_scatter(ch, idx_v, rows_v, sem):
            pltpu.make_async_copy(
                idx_hbm.at[pl.ds(ch * CHUNK, CHUNK)], idx_v, sem
            ).wait()
            pltpu.make_async_copy(
                e_hbm.at[pl.ds(ch * CHUNK, CHUNK)], rows_v, sem
            ).wait()
            pltpu.sync_copy(rows_v, acc_sh.at[idx_v], add=True)

        def zrow(i, _):
            def zcol(j, _):
                rows_a[i, pl.ds(j * 16, 16)] = jnp.zeros((16,), jnp.float32)
                return 0

            return lax.fori_loop(0, H // 16, zcol, 0)

        lax.fori_loop(0, CHUNK, zrow, 0)

        base = s * rows_per_tile
        nfull = rows_per_tile // CHUNK
        rem = rows_per_tile - nfull * CHUNK
        for i in range(nfull):
            pltpu.make_async_copy(
                rows_a, acc_sh.at[pl.ds(base + i * CHUNK, CHUNK)], sem_a
            ).start()
        pltpu.make_async_copy(
            rows_a.at[pl.ds(0, rem)],
            acc_sh.at[pl.ds(base + nfull * CHUNK, rem)],
            sem_a,
        ).start()
        for i in range(nfull):
            pltpu.make_async_copy(
                rows_a, acc_sh.at[pl.ds(base + i * CHUNK, CHUNK)], sem_a
            ).wait()
        pltpu.make_async_copy(
            rows_a.at[pl.ds(0, rem)],
            acc_sh.at[pl.ds(base + nfull * CHUNK, rem)],
            sem_a,
        ).wait()
        plsc.subcore_barrier()

        for b in range(NBUF):
            fetch(start + b, idx_bufs[b], rows_bufs[b], sems[b])

        def body(t, _):
            for b in range(NBUF):
                kk = NBUF * t + b
                kf = kk + NBUF

                @pl.when(kk < cnt)
                def _(b=b, kk=kk):
                    drain_and_scatter(
                        start + kk, idx_bufs[b], rows_bufs[b], sems[b]
                    )

                @pl.when(kf < cnt)
                def _(b=b, kf=kf):
                    fetch(start + kf, idx_bufs[b], rows_bufs[b], sems[b])

            return 0

        lax.fori_loop(0, n_trips, body, 0)
        plsc.subcore_barrier()

        pltpu.sync_copy(
            acc_sh.at[pl.ds(base, rows_per_tile)],
            out_hbm.at[c, pl.ds(base, rows_per_tile)],
        )

    return k(e, idx3)


def _tc_mlp(acc2, v, W1, b1, W2, b2):
    n = v.shape[0]
    blk = 5000

    def body(a0_ref, a1_ref, v_ref, w1_ref, b1_ref, w2_ref, b2_ref, o_ref):
        a = a0_ref[0] + a1_ref[0]
        h = jnp.dot(a, w1_ref[...], preferred_element_type=jnp.float32)
        h = jnp.maximum(h + b1_ref[...], 0.0)
        o = jnp.dot(h, w2_ref[...], preferred_element_type=jnp.float32)
        o_ref[...] = o + b2_ref[...] + v_ref[...]

    return pl.pallas_call(
        body,
        grid=(n // blk,),
        in_specs=[
            pl.BlockSpec((1, blk, H), lambda i: (0, i, 0)),
            pl.BlockSpec((1, blk, H), lambda i: (1, i, 0)),
            pl.BlockSpec((blk, H), lambda i: (i, 0)),
            pl.BlockSpec((H, H), lambda i: (0, 0)),
            pl.BlockSpec((1, H), lambda i: (0, 0)),
            pl.BlockSpec((H, H), lambda i: (0, 0)),
            pl.BlockSpec((1, H), lambda i: (0, 0)),
        ],
        out_specs=pl.BlockSpec((blk, H), lambda i: (i, 0)),
        out_shape=jax.ShapeDtypeStruct((n, H), jnp.float32),
    )(acc2, acc2, v, W1, b1.reshape(1, H), W2, b2.reshape(1, H))


def kernel(v, e, edge_index, W1, b1, W2, b2):
    n_pad = 10112
    idx3 = edge_index[1].astype(jnp.int32)
    acc2 = _sc_segment_sum(e, idx3, n_pad)
    return _tc_mlp(acc2, v, W1, b1, W2, b2)


# speedup vs baseline: 1.0950x; 1.0950x over previous
import functools

import jax
import jax.numpy as jnp
from jax import lax
from jax.experimental import pallas as pl
from jax.experimental.pallas import tpu as pltpu
from jax.experimental.pallas import tpu_sc as plsc

H = 128
CHUNK = 128
NBUF = 3


def _sc_segment_sum(e, idx3, n_pad):
    n_chunks = idx3.shape[0]
    info = plsc.get_sparse_core_info()
    nc, ns = info.num_cores, info.num_subcores
    nw = nc * ns
    rows_per_tile = n_pad // ns
    n_trips = (-(-n_chunks // nw) + NBUF - 1) // NBUF

    mesh = plsc.VectorSubcoreMesh(core_axis_name="c", subcore_axis_name="s")

    @functools.partial(
        pl.kernel,
        mesh=mesh,
        out_type=jax.ShapeDtypeStruct((nc, n_pad, H), jnp.float32),
        scratch_types=[
            pltpu.VMEM((1, CHUNK), jnp.int32),
            pltpu.VMEM((1, CHUNK), jnp.int32),
            pltpu.VMEM((1, CHUNK), jnp.int32),
            pltpu.VMEM((CHUNK, H), jnp.float32),
            pltpu.VMEM((CHUNK, H), jnp.float32),
            pltpu.VMEM((CHUNK, H), jnp.float32),
            pltpu.SemaphoreType.DMA,
            pltpu.SemaphoreType.DMA,
            pltpu.SemaphoreType.DMA,
            pltpu.SemaphoreType.DMA,
            pltpu.SemaphoreType.DMA,
            pltpu.SemaphoreType.DMA,
            pltpu.VMEM_SHARED((n_pad, H), jnp.float32),
        ],
    )
    def k(e_hbm, idx_hbm, out_hbm, idx_a, idx_b, idx_c, rows_a, rows_b,
          rows_c, sem_a, sem_b, sem_c, ssem_a, ssem_b, ssem_c, acc_sh):
        c = lax.axis_index("c")
        s = lax.axis_index("s")
        wid = s * nc + c
        cbase = n_chunks // nw
        cextra = n_chunks - cbase * nw
        start = wid * cbase + jnp.minimum(wid, cextra)
        cnt = cbase + jnp.where(wid < cextra, 1, 0)
        idx_bufs = [idx_a, idx_b, idx_c]
        rows_bufs = [rows_a, rows_b, rows_c]
        sems = [sem_a, sem_b, sem_c]
        ssems = [ssem_a, ssem_b, ssem_c]

        def fetch(ch, idx_v, rows_v, sem):
            pltpu.make_async_copy(idx_hbm.at[ch], idx_v, sem).start()
            pltpu.make_async_copy(
                e_hbm.at[pl.ds(ch * CHUNK, CHUNK)], rows_v, sem
            ).start()

        def drain_and_scatter(ch, idx_v, rows_v, sem, ssem):
            pltpu.make_async_copy(idx_hbm.at[ch], idx_v, sem).wait()
            pltpu.make_async_copy(
                e_hbm.at[pl.ds(ch * CHUNK, CHUNK)], rows_v, sem
            ).wait()
            pltpu.make_async_copy(
                rows_v, acc_sh.at[idx_v.at[0]], ssem
            ).start(add=True)

        def zrow(i, _):
            def zcol(j, _):
                rows_a[i, pl.ds(j * 16, 16)] = jnp.zeros((16,), jnp.float32)
                return 0

            return lax.fori_loop(0, H // 16, zcol, 0)

        lax.fori_loop(0, CHUNK, zrow, 0)

        base = s * rows_per_tile
        nfull = rows_per_tile // CHUNK
        rem = rows_per_tile - nfull * CHUNK
        for i in range(nfull):
            pltpu.make_async_copy(
                rows_a, acc_sh.at[pl.ds(base + i * CHUNK, CHUNK)], sem_a
            ).start()
        pltpu.make_async_copy(
            rows_a.at[pl.ds(0, rem)],
            acc_sh.at[pl.ds(base + nfull * CHUNK, rem)],
            sem_a,
        ).start()
        for i in range(nfull):
            pltpu.make_async_copy(
                rows_a, acc_sh.at[pl.ds(base + i * CHUNK, CHUNK)], sem_a
            ).wait()
        pltpu.make_async_copy(
            rows_a.at[pl.ds(0, rem)],
            acc_sh.at[pl.ds(base + nfull * CHUNK, rem)],
            sem_a,
        ).wait()
        plsc.subcore_barrier()

        for b in range(NBUF):
            fetch(start + b, idx_bufs[b], rows_bufs[b], sems[b])

        def scatter_wait(idx_v, rows_v, ssem):
            pltpu.make_async_copy(
                rows_v, acc_sh.at[idx_v.at[0]], ssem
            ).wait()

        def body(t, _):
            for b in range(NBUF):
                kk = NBUF * t + b
                kf = kk + NBUF

                @pl.when(kk < cnt)
                def _(b=b, kk=kk):
                    drain_and_scatter(
                        start + kk, idx_bufs[b], rows_bufs[b], sems[b],
                        ssems[b],
                    )

                @pl.when(kf < cnt)
                def _(b=b, kf=kf):
                    scatter_wait(idx_bufs[b], rows_bufs[b], ssems[b])
                    fetch(start + kf, idx_bufs[b], rows_bufs[b], sems[b])

            return 0

        lax.fori_loop(0, n_trips, body, 0)

        for b in range(NBUF):
            tail = cnt - NBUF + b

            @pl.when(tail >= 0)
            def _(b=b):
                scatter_wait(idx_bufs[b], rows_bufs[b], ssems[b])

        plsc.subcore_barrier()

        pltpu.sync_copy(
            acc_sh.at[pl.ds(base, rows_per_tile)],
            out_hbm.at[c, pl.ds(base, rows_per_tile)],
        )

    return k(e, idx3)


def _tc_mlp(acc2, v, W1, b1, W2, b2):
    n = v.shape[0]
    blk = 5000

    def body(a0_ref, a1_ref, v_ref, w1_ref, b1_ref, w2_ref, b2_ref, o_ref):
        a = a0_ref[0] + a1_ref[0]
        h = jnp.dot(a, w1_ref[...], preferred_element_type=jnp.float32)
        h = jnp.maximum(h + b1_ref[...], 0.0)
        o = jnp.dot(h, w2_ref[...], preferred_element_type=jnp.float32)
        o_ref[...] = o + b2_ref[...] + v_ref[...]

    return pl.pallas_call(
        body,
        grid=(n // blk,),
        in_specs=[
            pl.BlockSpec((1, blk, H), lambda i: (0, i, 0)),
            pl.BlockSpec((1, blk, H), lambda i: (1, i, 0)),
            pl.BlockSpec((blk, H), lambda i: (i, 0)),
            pl.BlockSpec((H, H), lambda i: (0, 0)),
            pl.BlockSpec((1, H), lambda i: (0, 0)),
            pl.BlockSpec((H, H), lambda i: (0, 0)),
            pl.BlockSpec((1, H), lambda i: (0, 0)),
        ],
        out_specs=pl.BlockSpec((blk, H), lambda i: (i, 0)),
        out_shape=jax.ShapeDtypeStruct((n, H), jnp.float32),
    )(acc2, acc2, v, W1, b1.reshape(1, H), W2, b2.reshape(1, H))


def kernel(v, e, edge_index, W1, b1, W2, b2):
    n_pad = 10112
    idx3 = edge_index[1].astype(jnp.int32).reshape(-1, 1, CHUNK)
    acc2 = _sc_segment_sum(e, idx3, n_pad)
    return _tc_mlp(acc2, v, W1, b1, W2, b2)


# speedup vs baseline: 1.1197x; 1.0225x over previous
import functools

import jax
import jax.numpy as jnp
from jax import lax
from jax.experimental import pallas as pl
from jax.experimental.pallas import tpu as pltpu
from jax.experimental.pallas import tpu_sc as plsc

H = 128
CHUNK = 128
NBUF = 3


def _sc_segment_sum(e, idx3, n_pad):
    n_chunks = idx3.shape[0]
    info = plsc.get_sparse_core_info()
    nc, ns = info.num_cores, info.num_subcores
    nw = nc * ns
    rows_per_tile = n_pad // ns
    n_trips = (-(-n_chunks // nw) + NBUF - 1) // NBUF

    mesh = plsc.VectorSubcoreMesh(core_axis_name="c", subcore_axis_name="s")

    @functools.partial(
        pl.kernel,
        mesh=mesh,
        out_type=jax.ShapeDtypeStruct((nc, n_pad, H), jnp.float32),
        scratch_types=[
            pltpu.VMEM((1, CHUNK), jnp.int32),
            pltpu.VMEM((1, CHUNK), jnp.int32),
            pltpu.VMEM((1, CHUNK), jnp.int32),
            pltpu.VMEM((CHUNK, H), jnp.float32),
            pltpu.VMEM((CHUNK, H), jnp.float32),
            pltpu.VMEM((CHUNK, H), jnp.float32),
            pltpu.SemaphoreType.DMA,
            pltpu.SemaphoreType.DMA,
            pltpu.SemaphoreType.DMA,
            pltpu.SemaphoreType.DMA,
            pltpu.SemaphoreType.DMA,
            pltpu.SemaphoreType.DMA,
            pltpu.VMEM_SHARED((n_pad, H), jnp.float32),
        ],
    )
    def k(e_hbm, idx_hbm, out_hbm, idx_a, idx_b, idx_c, rows_a, rows_b,
          rows_c, sem_a, sem_b, sem_c, ssem_a, ssem_b, ssem_c, acc_sh):
        c = lax.axis_index("c")
        s = lax.axis_index("s")
        wid = s * nc + c
        cbase = n_chunks // nw
        cextra = n_chunks - cbase * nw
        start = wid * cbase + jnp.minimum(wid, cextra)
        cnt = cbase + jnp.where(wid < cextra, 1, 0)
        idx_bufs = [idx_a, idx_b, idx_c]
        rows_bufs = [rows_a, rows_b, rows_c]
        sems = [sem_a, sem_b, sem_c]
        ssems = [ssem_a, ssem_b, ssem_c]

        def fetch(ch, idx_v, rows_v, sem):
            pltpu.make_async_copy(idx_hbm.at[ch], idx_v, sem).start()
            pltpu.make_async_copy(
                e_hbm.at[pl.ds(ch * CHUNK, CHUNK)], rows_v, sem
            ).start()

        def drain_and_scatter(ch, idx_v, rows_v, sem, ssem):
            pltpu.make_async_copy(idx_hbm.at[ch], idx_v, sem).wait()
            pltpu.make_async_copy(
                e_hbm.at[pl.ds(ch * CHUNK, CHUNK)], rows_v, sem
            ).wait()
            pltpu.make_async_copy(
                rows_v, acc_sh.at[idx_v.at[0]], ssem
            ).start(add=True)

        for b in range(2):
            fetch(start + b, idx_bufs[b], rows_bufs[b], sems[b])

        ZR = 32

        def zrow(i, _):
            def zcol(j, _):
                rows_c[i, pl.ds(j * 16, 16)] = jnp.zeros((16,), jnp.float32)
                return 0

            return lax.fori_loop(0, H // 16, zcol, 0)

        lax.fori_loop(0, ZR, zrow, 0)

        base = s * rows_per_tile
        nfull = rows_per_tile // ZR
        rem = rows_per_tile - nfull * ZR

        def zcopy(i):
            return pltpu.make_async_copy(
                rows_c.at[pl.ds(0, ZR)],
                acc_sh.at[pl.ds(base + i * ZR, ZR)],
                ssem_c,
            )

        def zcopy_rem():
            return pltpu.make_async_copy(
                rows_c.at[pl.ds(0, rem)],
                acc_sh.at[pl.ds(base + nfull * ZR, rem)],
                ssem_c,
            )

        for i in range(nfull):
            zcopy(i).start()
        if rem:
            zcopy_rem().start()
        for i in range(nfull):
            zcopy(i).wait()
        if rem:
            zcopy_rem().wait()

        fetch(start + 2, idx_bufs[2], rows_bufs[2], sems[2])
        plsc.subcore_barrier()

        def scatter_wait(idx_v, rows_v, ssem):
            pltpu.make_async_copy(
                rows_v, acc_sh.at[idx_v.at[0]], ssem
            ).wait()

        def body(t, _):
            for b in range(NBUF):
                kk = NBUF * t + b
                kf = kk + NBUF

                @pl.when(kk < cnt)
                def _(b=b, kk=kk):
                    drain_and_scatter(
                        start + kk, idx_bufs[b], rows_bufs[b], sems[b],
                        ssems[b],
                    )

                @pl.when(kf < cnt)
                def _(b=b, kf=kf):
                    scatter_wait(idx_bufs[b], rows_bufs[b], ssems[b])
                    fetch(start + kf, idx_bufs[b], rows_bufs[b], sems[b])

            return 0

        lax.fori_loop(0, n_trips, body, 0)

        for b in range(NBUF):
            tail = cnt - NBUF + b

            @pl.when(tail >= 0)
            def _(b=b):
                scatter_wait(idx_bufs[b], rows_bufs[b], ssems[b])

        plsc.subcore_barrier()

        pltpu.sync_copy(
            acc_sh.at[pl.ds(base, rows_per_tile)],
            out_hbm.at[c, pl.ds(base, rows_per_tile)],
        )

    return k(e, idx3)


def _tc_mlp(acc2, v, W1, b1, W2, b2):
    n = v.shape[0]
    blk = 5000

    def body(a0_ref, a1_ref, v_ref, w1_ref, b1_ref, w2_ref, b2_ref, o_ref):
        a = a0_ref[0] + a1_ref[0]
        h = jnp.dot(a, w1_ref[...], preferred_element_type=jnp.float32)
        h = jnp.maximum(h + b1_ref[...], 0.0)
        o = jnp.dot(h, w2_ref[...], preferred_element_type=jnp.float32)
        o_ref[...] = o + b2_ref[...] + v_ref[...]

    return pl.pallas_call(
        body,
        grid=(n // blk,),
        in_specs=[
            pl.BlockSpec((1, blk, H), lambda i: (0, i, 0)),
            pl.BlockSpec((1, blk, H), lambda i: (1, i, 0)),
            pl.BlockSpec((blk, H), lambda i: (i, 0)),
            pl.BlockSpec((H, H), lambda i: (0, 0)),
            pl.BlockSpec((1, H), lambda i: (0, 0)),
            pl.BlockSpec((H, H), lambda i: (0, 0)),
            pl.BlockSpec((1, H), lambda i: (0, 0)),
        ],
        out_specs=pl.BlockSpec((blk, H), lambda i: (i, 0)),
        out_shape=jax.ShapeDtypeStruct((n, H), jnp.float32),
    )(acc2, acc2, v, W1, b1.reshape(1, H), W2, b2.reshape(1, H))


def kernel(v, e, edge_index, W1, b1, W2, b2):
    n_pad = 10112
    idx3 = edge_index[1].astype(jnp.int32).reshape(-1, 1, CHUNK)
    acc2 = _sc_segment_sum(e, idx3, n_pad)
    return _tc_mlp(acc2, v, W1, b1, W2, b2)
